# Initial kernel scaffold; baseline (speedup 1.0000x reference)
import jax
import jax.numpy as jnp
from jax.experimental import pallas as pl


def kernel(x, w1, b1, g1, be1, w2, b2):
    raise NotImplementedError("write your pallas kernel here")


# speedup vs baseline: 1.8584x; 1.8584x over previous
import functools

import jax
import jax.numpy as jnp
from jax.experimental import pallas as pl
from jax.experimental.pallas import tpu as pltpu

_EPS = 1e-5
_LANE = 128


def _ru(v, m):
    return (v + m - 1) // m * m


def _row_mask(c, t, bm, t_steps, m):
    row0 = (c * t_steps + t) * bm
    row = row0 + jax.lax.broadcasted_iota(jnp.int32, (bm, 1), 0)
    return (row < m).astype(jnp.float32)


def _pass1_kernel(x_ref, w1_ref, b1_ref, xb_ref, p1_ref,
                  *, bm, t_steps, m, masked):
    c = pl.program_id(0)
    t = pl.program_id(1)

    @pl.when(t == 0)
    def _():
        p1_ref[...] = jnp.zeros_like(p1_ref)

    xb = x_ref[...].astype(jnp.bfloat16)
    xb_ref[...] = xb
    h = jnp.dot(xb, w1_ref[...], preferred_element_type=jnp.float32)
    h = h + b1_ref[...]
    if masked:
        hv = h * _row_mask(c, t, bm, t_steps, m)
    else:
        hv = h
    p1_ref[0, 0:1, :] += jnp.sum(hv, axis=0, keepdims=True)
    p1_ref[0, 1:2, :] += jnp.sum(hv * h, axis=0, keepdims=True)


def _combine_stats(p_ref, inv_m):
    p = p_ref[...]
    s = p[0] + p[1]
    mean = s[0:1] * inv_m
    var = jnp.maximum(s[1:2] * inv_m - mean * mean, 0.0)
    return mean, jax.lax.rsqrt(var + _EPS)


def _chain(xb_ref, w1_ref, b1_ref, g1_ref, be1_ref, p1_ref, w2_ref, b2_ref,
           inv_m):
    mean1, rstd1 = _combine_stats(p1_ref, inv_m)
    scale1 = rstd1 * g1_ref[...]
    shift1 = (b1_ref[...] - mean1) * scale1 + be1_ref[...]
    h = jnp.dot(xb_ref[...], w1_ref[...], preferred_element_type=jnp.float32)
    a = jnp.maximum(h * scale1 + shift1, 0.0)
    return jnp.dot(a.astype(jnp.bfloat16), w2_ref[...],
                   preferred_element_type=jnp.float32) + b2_ref[...]


def _pass2_kernel(xb_ref, w1_ref, b1_ref, g1_ref, be1_ref, p1_ref,
                  w2_ref, b2_ref, p2_ref, *, bm, t_steps, m, inv_m, masked):
    c = pl.program_id(0)
    t = pl.program_id(1)

    @pl.when(t == 0)
    def _():
        p2_ref[...] = jnp.zeros_like(p2_ref)

    y = _chain(xb_ref, w1_ref, b1_ref, g1_ref, be1_ref, p1_ref,
               w2_ref, b2_ref, inv_m)
    if masked:
        yv = y * _row_mask(c, t, bm, t_steps, m)
    else:
        yv = y
    p2_ref[0, 0:1, :] += jnp.sum(yv, axis=0, keepdims=True)
    p2_ref[0, 1:2, :] += jnp.sum(yv * y, axis=0, keepdims=True)


def _pass3_kernel(xb_ref, w1_ref, b1_ref, g1_ref, be1_ref, p1_ref,
                  w2_ref, b2_ref, p2_ref, o_ref, *, inv_m):
    y = _chain(xb_ref, w1_ref, b1_ref, g1_ref, be1_ref, p1_ref,
               w2_ref, b2_ref, inv_m)
    mean2, rstd2 = _combine_stats(p2_ref, inv_m)
    o_ref[...] = (y - mean2) * rstd2


def kernel(x, w1, b1, g1, be1, w2, b2):
    B, N, C = x.shape
    H = w1.shape[1]
    O = w2.shape[1]
    M = B * N

    Cp = _ru(C, _LANE)
    Hp = _ru(H, _LANE)
    Op = _ru(O, _LANE)
    w1b = jnp.zeros((Cp, Hp), jnp.bfloat16).at[:C, :H].set(w1.astype(jnp.bfloat16))
    b1p = jnp.zeros((1, Hp), jnp.float32).at[:, :H].set(b1)
    g1p = jnp.ones((1, Hp), jnp.float32).at[:, :H].set(g1)
    be1p = jnp.zeros((1, Hp), jnp.float32).at[:, :H].set(be1)
    w2b = jnp.zeros((Hp, Op), jnp.bfloat16).at[:H, :O].set(w2.astype(jnp.bfloat16))
    b2p = jnp.zeros((1, Op), jnp.float32).at[:, :O].set(b2)

    bm = min(4096, _ru(M, 16))
    t_steps = -(-M // (2 * bm))
    Mp = 2 * t_steps * bm
    masked = Mp != M

    x2d = x.reshape(M, C).astype(jnp.float32)
    if Mp != M or Cp != C:
        x2d = jnp.zeros((Mp, Cp), jnp.float32).at[:M, :C].set(x2d)

    row_spec = lambda ch: pl.BlockSpec((bm, ch), lambda c, t: (c * t_steps + t, 0))

    def full(a):
        return pl.BlockSpec(a.shape, lambda c, t: (0,) * a.ndim)

    p1_spec = pl.BlockSpec((1, 2, Hp), lambda c, t: (c, 0, 0))
    p2_spec = pl.BlockSpec((1, 2, Op), lambda c, t: (c, 0, 0))

    grid = (2, t_steps)
    cp = pltpu.CompilerParams(dimension_semantics=("parallel", "arbitrary"),
                              vmem_limit_bytes=48 * 1024 * 1024)
    inv_m = 1.0 / M

    xb, part1 = pl.pallas_call(
        functools.partial(_pass1_kernel, bm=bm, t_steps=t_steps, m=M,
                          masked=masked),
        out_shape=(jax.ShapeDtypeStruct((Mp, Cp), jnp.bfloat16),
                   jax.ShapeDtypeStruct((2, 2, Hp), jnp.float32)),
        grid=grid,
        in_specs=[row_spec(Cp), full(w1b), full(b1p)],
        out_specs=(row_spec(Cp), p1_spec),
        compiler_params=cp,
    )(x2d, w1b, b1p)

    part2 = pl.pallas_call(
        functools.partial(_pass2_kernel, bm=bm, t_steps=t_steps, m=M,
                          inv_m=inv_m, masked=masked),
        out_shape=jax.ShapeDtypeStruct((2, 2, Op), jnp.float32),
        grid=grid,
        in_specs=[row_spec(Cp), full(w1b), full(b1p), full(g1p), full(be1p),
                  full(part1), full(w2b), full(b2p)],
        out_specs=p2_spec,
        compiler_params=cp,
    )(xb, w1b, b1p, g1p, be1p, part1, w2b, b2p)

    out_p = pl.pallas_call(
        functools.partial(_pass3_kernel, inv_m=inv_m),
        out_shape=jax.ShapeDtypeStruct((Mp, Op), jnp.float32),
        grid=grid,
        in_specs=[row_spec(Cp), full(w1b), full(b1p), full(g1p), full(be1p),
                  full(part1), full(w2b), full(b2p), full(part2)],
        out_specs=row_spec(Op),
        compiler_params=cp,
    )(xb, w1b, b1p, g1p, be1p, part1, w2b, b2p, part2)

    return out_p[:M, :O].reshape(B, N, O)


# speedup vs baseline: 2.0892x; 1.1242x over previous
import functools

import jax
import jax.numpy as jnp
from jax.experimental import pallas as pl
from jax.experimental.pallas import tpu as pltpu

_EPS = 1e-5
_LANE = 128


def _ru(v, m):
    return (v + m - 1) // m * m


def _fused_kernel(x_ref, w1_ref, b1_ref, g1_ref, be1_ref, w2_ref, b2_ref,
                  o_ref,
                  xb_ref, gram_ref, cs1_ref, w1s_ref, sh1_ref,
                  zs_ref, zq_ref, w2s_ref, c2_ref,
                  *, bm, t_steps, m, masked):
    p = pl.program_id(0)
    t = pl.program_id(1)
    inv_m = jnp.float32(1.0 / m)
    rows = pl.ds(t * bm, bm)

    @pl.when(p == 0)
    def _phase0():
        @pl.when(t == 0)
        def _():
            gram_ref[...] = jnp.zeros_like(gram_ref)
            cs1_ref[...] = jnp.zeros_like(cs1_ref)

        x = x_ref[...]
        xb = x.astype(jnp.bfloat16)
        xb_ref[rows, :] = xb
        gram_ref[...] += jax.lax.dot_general(
            xb, xb, (((0,), (0,)), ((), ())),
            preferred_element_type=jnp.float32)
        cs1_ref[...] += jnp.sum(x, axis=0, keepdims=True)

    @pl.when(jnp.logical_and(p == 1, t == 0))
    def _stats1():
        w1b = w1_ref[...]
        w1f = w1b.astype(jnp.float32)
        cs = cs1_ref[...]
        sh0 = jnp.dot(cs.astype(jnp.bfloat16), w1b,
                      preferred_element_type=jnp.float32)
        d = jnp.dot(gram_ref[...].astype(jnp.bfloat16), w1b,
                    preferred_element_type=jnp.float32)
        sq0 = jnp.sum(w1f * d, axis=0, keepdims=True)
        b1 = b1_ref[...]
        mean1 = sh0 * inv_m + b1
        ex2 = (sq0 + 2.0 * b1 * sh0) * inv_m + b1 * b1
        var1 = jnp.maximum(ex2 - mean1 * mean1, 0.0)
        scale1 = g1_ref[...] * jax.lax.rsqrt(var1 + _EPS)
        sh1_ref[...] = (b1 - mean1) * scale1 + be1_ref[...]
        w1s_ref[...] = (w1f * scale1).astype(jnp.bfloat16)
        zs_ref[...] = jnp.zeros_like(zs_ref)
        zq_ref[...] = jnp.zeros_like(zq_ref)

    @pl.when(p == 1)
    def _phase1():
        xb = xb_ref[rows, :]
        hs = jnp.dot(xb, w1s_ref[...], preferred_element_type=jnp.float32)
        a = jnp.maximum(hs + sh1_ref[...], 0.0)
        z = jnp.dot(a.astype(jnp.bfloat16), w2_ref[...],
                    preferred_element_type=jnp.float32)
        if masked:
            row = t * bm + jax.lax.broadcasted_iota(jnp.int32, (bm, 1), 0)
            z = z * (row < m).astype(jnp.float32)
        zs_ref[...] += jnp.sum(z, axis=0, keepdims=True)
        zq_ref[...] += jnp.sum(z * z, axis=0, keepdims=True)

    @pl.when(jnp.logical_and(p == 2, t == 0))
    def _stats2():
        mz = zs_ref[...] * inv_m
        vz = jnp.maximum(zq_ref[...] * inv_m - mz * mz, 0.0)
        rstd2 = jax.lax.rsqrt(vz + _EPS)
        w2s_ref[...] = (w2_ref[...].astype(jnp.float32) * rstd2
                        ).astype(jnp.bfloat16)
        c2_ref[...] = -mz * rstd2

    @pl.when(p == 2)
    def _phase2():
        xb = xb_ref[rows, :]
        hs = jnp.dot(xb, w1s_ref[...], preferred_element_type=jnp.float32)
        a = jnp.maximum(hs + sh1_ref[...], 0.0)
        z = jnp.dot(a.astype(jnp.bfloat16), w2s_ref[...],
                    preferred_element_type=jnp.float32)
        o_ref[...] = z + c2_ref[...]


def kernel(x, w1, b1, g1, be1, w2, b2):
    B, N, C = x.shape
    H = w1.shape[1]
    O = w2.shape[1]
    M = B * N

    Cp = _ru(C, _LANE)
    Hp = _ru(H, _LANE)
    Op = _ru(O, _LANE)
    w1b = jnp.zeros((Cp, Hp), jnp.bfloat16).at[:C, :H].set(w1.astype(jnp.bfloat16))
    b1p = jnp.zeros((1, Hp), jnp.float32).at[:, :H].set(b1)
    g1p = jnp.ones((1, Hp), jnp.float32).at[:, :H].set(g1)
    be1p = jnp.zeros((1, Hp), jnp.float32).at[:, :H].set(be1)
    w2b = jnp.zeros((Hp, Op), jnp.bfloat16).at[:H, :O].set(w2.astype(jnp.bfloat16))
    b2p = jnp.zeros((1, Op), jnp.float32).at[:, :O].set(b2)

    bm = min(4096, _ru(M, 16))
    t_steps = -(-M // bm)
    Mp = t_steps * bm
    masked = Mp != M

    x2d = x.reshape(M, C).astype(jnp.float32)
    if Mp != M or Cp != C:
        x2d = jnp.zeros((Mp, Cp), jnp.float32).at[:M, :C].set(x2d)

    def full(a):
        return pl.BlockSpec(a.shape, lambda p, t: (0,) * a.ndim)

    x_spec = pl.BlockSpec((bm, Cp), lambda p, t: (jnp.where(p == 0, t, 0), 0))
    o_spec = pl.BlockSpec((bm, Op), lambda p, t: (jnp.where(p == 2, t, 0), 0))

    out_p = pl.pallas_call(
        functools.partial(_fused_kernel, bm=bm, t_steps=t_steps, m=M,
                          masked=masked),
        out_shape=jax.ShapeDtypeStruct((Mp, Op), jnp.float32),
        grid=(3, t_steps),
        in_specs=[x_spec, full(w1b), full(b1p), full(g1p), full(be1p),
                  full(w2b), full(b2p)],
        out_specs=o_spec,
        scratch_shapes=[
            pltpu.VMEM((Mp, Cp), jnp.bfloat16),
            pltpu.VMEM((Cp, Cp), jnp.float32),
            pltpu.VMEM((1, Cp), jnp.float32),
            pltpu.VMEM((Cp, Hp), jnp.bfloat16),
            pltpu.VMEM((1, Hp), jnp.float32),
            pltpu.VMEM((1, Op), jnp.float32),
            pltpu.VMEM((1, Op), jnp.float32),
            pltpu.VMEM((Hp, Op), jnp.bfloat16),
            pltpu.VMEM((1, Op), jnp.float32),
        ],
        compiler_params=pltpu.CompilerParams(
            dimension_semantics=("arbitrary", "arbitrary"),
            vmem_limit_bytes=48 * 1024 * 1024),
    )(x2d, w1b, b1p, g1p, be1p, w2b, b2p)

    return out_p[:M, :O].reshape(B, N, O)


# speedup vs baseline: 2.2850x; 1.0937x over previous
import functools

import jax
import jax.numpy as jnp
from jax.experimental import pallas as pl
from jax.experimental.pallas import tpu as pltpu

_EPS = 1e-5
_LANE = 128


def _ru(v, m):
    return (v + m - 1) // m * m


def _rowsum8(v):
    r = v.shape[0]
    while r > 8 and r % 2 == 0:
        half = r // 2
        v = v[:half] + v[half:]
        r = half
    if r > 8:
        v = jnp.concatenate(
            [jnp.sum(v, axis=0, keepdims=True),
             jnp.zeros((7, v.shape[1]), v.dtype)], axis=0)
    return v


def _fused_kernel(x_ref, w1_ref, b1_ref, g1_ref, be1_ref, w2_ref, b2_ref,
                  o_ref,
                  xb_ref, gram_ref, cs1_ref, w1s_ref, sh1_ref,
                  zs_ref, zq_ref, w2s_ref, c2_ref,
                  *, bm, t_steps, m, masked):
    p = pl.program_id(0)
    t = pl.program_id(1)
    inv_m = jnp.float32(1.0 / m)
    rows = pl.ds(t * bm, bm)

    @pl.when(p == 0)
    def _phase0():
        @pl.when(t == 0)
        def _():
            gram_ref[...] = jnp.zeros_like(gram_ref)
            cs1_ref[...] = jnp.zeros_like(cs1_ref)

        x = x_ref[...]
        xb = x.astype(jnp.bfloat16)
        xb_ref[rows, :] = xb
        gram_ref[...] += jax.lax.dot_general(
            xb, xb, (((0,), (0,)), ((), ())),
            preferred_element_type=jnp.float32)
        cs1_ref[...] += _rowsum8(x)

    @pl.when(jnp.logical_and(p == 1, t == 0))
    def _stats1():
        w1b = w1_ref[...]
        w1f = w1b.astype(jnp.float32)
        cs = jnp.sum(cs1_ref[...], axis=0, keepdims=True)
        sh0 = jnp.dot(cs.astype(jnp.bfloat16), w1b,
                      preferred_element_type=jnp.float32)
        d = jnp.dot(gram_ref[...].astype(jnp.bfloat16), w1b,
                    preferred_element_type=jnp.float32)
        sq0 = jnp.sum(_rowsum8(w1f * d), axis=0, keepdims=True)
        b1 = b1_ref[...]
        mean1 = sh0 * inv_m + b1
        ex2 = (sq0 + 2.0 * b1 * sh0) * inv_m + b1 * b1
        var1 = jnp.maximum(ex2 - mean1 * mean1, 0.0)
        scale1 = g1_ref[...] * jax.lax.rsqrt(var1 + _EPS)
        sh1_ref[...] = (b1 - mean1) * scale1 + be1_ref[...]
        w1s_ref[...] = (w1f * scale1).astype(jnp.bfloat16)
        zs_ref[...] = jnp.zeros_like(zs_ref)
        zq_ref[...] = jnp.zeros_like(zq_ref)

    @pl.when(p == 1)
    def _phase1():
        xb = xb_ref[rows, :]
        hs = jnp.dot(xb, w1s_ref[...], preferred_element_type=jnp.float32)
        a = jnp.maximum(hs + sh1_ref[...], 0.0)
        z = jnp.dot(a.astype(jnp.bfloat16), w2_ref[...],
                    preferred_element_type=jnp.float32)
        if masked:
            row = t * bm + jax.lax.broadcasted_iota(jnp.int32, (bm, 1), 0)
            z = z * (row < m).astype(jnp.float32)
        zs_ref[...] += _rowsum8(z)
        zq_ref[...] += _rowsum8(z * z)

    @pl.when(jnp.logical_and(p == 2, t == 0))
    def _stats2():
        mz = jnp.sum(zs_ref[...], axis=0, keepdims=True) * inv_m
        vz = jnp.maximum(
            jnp.sum(zq_ref[...], axis=0, keepdims=True) * inv_m - mz * mz, 0.0)
        rstd2 = jax.lax.rsqrt(vz + _EPS)
        w2s_ref[...] = (w2_ref[...].astype(jnp.float32) * rstd2
                        ).astype(jnp.bfloat16)
        c2_ref[...] = -mz * rstd2

    @pl.when(p == 2)
    def _phase2():
        xb = xb_ref[rows, :]
        hs = jnp.dot(xb, w1s_ref[...], preferred_element_type=jnp.float32)
        a = jnp.maximum(hs + sh1_ref[...], 0.0)
        z = jnp.dot(a.astype(jnp.bfloat16), w2s_ref[...],
                    preferred_element_type=jnp.float32)
        o_ref[...] = z + c2_ref[...]


def kernel(x, w1, b1, g1, be1, w2, b2):
    B, N, C = x.shape
    H = w1.shape[1]
    O = w2.shape[1]
    M = B * N

    Cp = _ru(C, _LANE)
    Hp = _ru(H, _LANE)
    Op = _ru(O, _LANE)
    w1b = jnp.zeros((Cp, Hp), jnp.bfloat16).at[:C, :H].set(w1.astype(jnp.bfloat16))
    b1p = jnp.zeros((1, Hp), jnp.float32).at[:, :H].set(b1)
    g1p = jnp.ones((1, Hp), jnp.float32).at[:, :H].set(g1)
    be1p = jnp.zeros((1, Hp), jnp.float32).at[:, :H].set(be1)
    w2b = jnp.zeros((Hp, Op), jnp.bfloat16).at[:H, :O].set(w2.astype(jnp.bfloat16))
    b2p = jnp.zeros((1, Op), jnp.float32).at[:, :O].set(b2)

    bm = min(4096, max(16, 1 << (M - 1).bit_length()))
    t_steps = -(-M // bm)
    Mp = t_steps * bm
    masked = Mp != M

    x2d = x.reshape(M, C).astype(jnp.float32)
    if Mp != M or Cp != C:
        x2d = jnp.zeros((Mp, Cp), jnp.float32).at[:M, :C].set(x2d)

    def full(a):
        return pl.BlockSpec(a.shape, lambda p, t: (0,) * a.ndim)

    x_spec = pl.BlockSpec((bm, Cp), lambda p, t: (jnp.where(p == 0, t, 0), 0))
    o_spec = pl.BlockSpec((bm, Op), lambda p, t: (jnp.where(p == 2, t, 0), 0))

    out_p = pl.pallas_call(
        functools.partial(_fused_kernel, bm=bm, t_steps=t_steps, m=M,
                          masked=masked),
        out_shape=jax.ShapeDtypeStruct((Mp, Op), jnp.float32),
        grid=(3, t_steps),
        in_specs=[x_spec, full(w1b), full(b1p), full(g1p), full(be1p),
                  full(w2b), full(b2p)],
        out_specs=o_spec,
        scratch_shapes=[
            pltpu.VMEM((Mp, Cp), jnp.bfloat16),
            pltpu.VMEM((Cp, Cp), jnp.float32),
            pltpu.VMEM((8, Cp), jnp.float32),
            pltpu.VMEM((Cp, Hp), jnp.bfloat16),
            pltpu.VMEM((1, Hp), jnp.float32),
            pltpu.VMEM((8, Op), jnp.float32),
            pltpu.VMEM((8, Op), jnp.float32),
            pltpu.VMEM((Hp, Op), jnp.bfloat16),
            pltpu.VMEM((1, Op), jnp.float32),
        ],
        compiler_params=pltpu.CompilerParams(
            dimension_semantics=("arbitrary", "arbitrary"),
            vmem_limit_bytes=48 * 1024 * 1024),
    )(x2d, w1b, b1p, g1p, be1p, w2b, b2p)

    return out_p[:M, :O].reshape(B, N, O)
